# Initial kernel scaffold; baseline (speedup 1.0000x reference)
import jax
import jax.numpy as jnp
from jax.experimental import pallas as pl


def kernel(class_logits, proposals, W1s, b1s, W2s, b2s, W1o, b1o, W2o, b2o):
    raise NotImplementedError("write your pallas kernel here")


# speedup vs baseline: 312.1712x; 312.1712x over previous
import functools

import jax
import jax.numpy as jnp
from jax.experimental import pallas as pl
from jax.experimental.pallas import tpu as pltpu

_N = 4096
_C = 151
_H = 64
_TAKE = 64
_NBLK = 8
_BW = _N // _NBLK
_NEG = float("-inf")
_IBIG = 1 << 30


def _mlp(x, W1, b1, W2, b2):
    h = jnp.maximum(jax.lax.dot_general(
        x, W1, (((1,), (0,)), ((), ())),
        preferred_element_type=jnp.float32) + b1, 0.0)
    return jax.lax.dot_general(
        h, W2, (((1,), (0,)), ((), ())),
        preferred_element_type=jnp.float32) + b2


def _relpn_kernel(x_ref, W1s_ref, b1s_ref, W2s_ref, b2s_ref,
                  W1o_ref, b1o_ref, W2o_ref, b2o_ref,
                  pairs_ref, scores_ref, subj_ref, L_ref):
    x = x_ref[0]
    subj = _mlp(x, W1s_ref[...], b1s_ref[...], W2s_ref[...], b2s_ref[...])
    obj = _mlp(x, W1o_ref[...], b1o_ref[...], W2o_ref[...], b2o_ref[...])
    subj_ref[...] = subj

    parts = []
    for blk in range(_NBLK):
        sblk = subj[blk * _BW:(blk + 1) * _BW, :]
        lbT = jax.lax.dot_general(
            obj, sblk, (((1,), (1,)), ((), ())),
            preferred_element_type=jnp.float32)
        parts.append(jnp.max(lbT, axis=0, keepdims=True))
    rowmax = jnp.concatenate(parts, axis=0)

    rowid = (jax.lax.broadcasted_iota(jnp.int32, (_NBLK, _BW), 0) * _BW
             + jax.lax.broadcasted_iota(jnp.int32, (_NBLK, _BW), 1))
    lane64 = jax.lax.broadcasted_iota(jnp.int32, (1, _TAKE), 1)
    sub64 = jax.lax.broadcasted_iota(jnp.int32, (_TAKE, 1), 0)

    def select_row(t, carry):
        rm, sel_lane, sel_col, gath = carry
        m = jnp.max(rm)
        r = jnp.min(jnp.where(rm == m, rowid, _IBIG))
        rm = jnp.where(rowid == r, _NEG, rm)
        sel_lane = jnp.where(lane64 == t, r, sel_lane)
        sel_col = jnp.where(sub64 == t, r, sel_col)
        row = subj_ref[pl.ds(r, 1), :]
        gath = jnp.where(sub64 == t, row, gath)
        return rm, sel_lane, sel_col, gath

    carry = (rowmax,
             jnp.zeros((1, _TAKE), jnp.int32),
             jnp.zeros((_TAKE, 1), jnp.int32),
             jnp.zeros((_TAKE, _H), jnp.float32))
    _, sel_lane, sel_col, gath = jax.lax.fori_loop(0, _TAKE, select_row, carry)

    L = jax.lax.dot_general(gath, obj, (((1,), (1,)), ((), ())),
                            preferred_element_type=jnp.float32)
    L_ref[...] = L
    lrowmax = jnp.max(L, axis=1, keepdims=True)
    col_iota = jax.lax.broadcasted_iota(jnp.int32, (1, _N), 1)

    def pop(t, carry):
        lrm, scores, rvec, cvec = carry
        m = jnp.max(lrm)
        ro = jnp.min(jnp.where(lrm == m, sel_col, _IBIG))
        tloc = jnp.min(jnp.where((lrm == m) & (sel_col == ro), sub64, _IBIG))
        lrow = L_ref[pl.ds(tloc, 1), :]
        c = jnp.min(jnp.where(lrow == m, col_iota, _IBIG))
        lrow2 = jnp.where(col_iota == c, _NEG, lrow)
        L_ref[pl.ds(tloc, 1), :] = lrow2
        lrm = jnp.where(sub64 == tloc, jnp.max(lrow2), lrm)
        scores = jnp.where(lane64 == t, m, scores)
        rvec = jnp.where(lane64 == t, ro, rvec)
        cvec = jnp.where(lane64 == t, c, cvec)
        return lrm, scores, rvec, cvec

    carry = (lrowmax,
             jnp.full((1, _TAKE), _NEG, jnp.float32),
             jnp.zeros((1, _TAKE), jnp.int32),
             jnp.zeros((1, _TAKE), jnp.int32))
    _, scores, rvec, cvec = jax.lax.fori_loop(0, _TAKE, pop, carry)

    pairs_ref[0, 0:1, :] = rvec
    pairs_ref[0, 1:2, :] = cvec
    scores_ref[0, 0:1, :] = jax.nn.sigmoid(scores)


@jax.jit
def kernel(class_logits, proposals, W1s, b1s, W2s, b2s, W1o, b1o, W2o, b2o):
    del proposals
    B = class_logits.shape[0]
    b1s2 = b1s.reshape(1, _H)
    b2s2 = b2s.reshape(1, _H)
    b1o2 = b1o.reshape(1, _H)
    b2o2 = b2o.reshape(1, _H)

    full = lambda shape: pl.BlockSpec(shape, lambda b: (0,) * len(shape))
    pairs2, scores = pl.pallas_call(
        _relpn_kernel,
        grid=(B,),
        in_specs=[
            pl.BlockSpec((1, _N, _C), lambda b: (b, 0, 0)),
            full((_C, _H)), full((1, _H)), full((_H, _H)), full((1, _H)),
            full((_C, _H)), full((1, _H)), full((_H, _H)), full((1, _H)),
        ],
        out_specs=[
            pl.BlockSpec((1, 2, _TAKE), lambda b: (b, 0, 0)),
            pl.BlockSpec((1, 1, _TAKE), lambda b: (b, 0, 0)),
        ],
        out_shape=[
            jax.ShapeDtypeStruct((B, 2, _TAKE), jnp.int32),
            jax.ShapeDtypeStruct((B, 1, _TAKE), jnp.float32),
        ],
        scratch_shapes=[
            pltpu.VMEM((_N, _H), jnp.float32),
            pltpu.VMEM((_TAKE, _N), jnp.float32),
        ],
        compiler_params=pltpu.CompilerParams(
            dimension_semantics=("arbitrary",),
        ),
    )(class_logits, W1s, b1s2, W2s, b2s2, W1o, b1o2, W2o, b2o2)

    pairs = jnp.swapaxes(pairs2, 1, 2)
    return pairs, scores[:, 0, :]


# speedup vs baseline: 312.1806x; 1.0000x over previous
import functools

import jax
import jax.numpy as jnp
from jax.experimental import pallas as pl
from jax.experimental.pallas import tpu as pltpu

_N = 4096
_C = 151
_H = 64
_TAKE = 64
_NBLK = 8
_BW = _N // _NBLK
_NEG = float("-inf")
_IBIG = 1 << 30


def _mlp(x, W1, b1, W2, b2):
    h = jnp.maximum(jax.lax.dot_general(
        x, W1, (((1,), (0,)), ((), ())),
        preferred_element_type=jnp.float32) + b1, 0.0)
    return jax.lax.dot_general(
        h, W2, (((1,), (0,)), ((), ())),
        preferred_element_type=jnp.float32) + b2


def _relpn_kernel(x_ref, W1s_ref, b1s_ref, W2s_ref, b2s_ref,
                  W1o_ref, b1o_ref, W2o_ref, b2o_ref,
                  pairs_ref, scores_ref, subj_ref, L_ref):
    x = x_ref[0]
    subj = _mlp(x, W1s_ref[...], b1s_ref[...], W2s_ref[...], b2s_ref[...])
    obj = _mlp(x, W1o_ref[...], b1o_ref[...], W2o_ref[...], b2o_ref[...])
    subj_ref[...] = subj

    parts = []
    for blk in range(_NBLK):
        sblk = subj[blk * _BW:(blk + 1) * _BW, :]
        lbT = jax.lax.dot_general(
            obj, sblk, (((1,), (1,)), ((), ())),
            preferred_element_type=jnp.float32)
        parts.append(jnp.max(lbT, axis=0, keepdims=True))
    rowmax = jnp.concatenate(parts, axis=0)

    rowid = (jax.lax.broadcasted_iota(jnp.int32, (_NBLK, _BW), 0) * _BW
             + jax.lax.broadcasted_iota(jnp.int32, (_NBLK, _BW), 1))
    lane64 = jax.lax.broadcasted_iota(jnp.int32, (1, _TAKE), 1)
    sub64 = jax.lax.broadcasted_iota(jnp.int32, (_TAKE, 1), 0)

    def select_row(t, carry):
        rm, sel_lane, sel_col, gath = carry
        m = jnp.max(rm)
        r = jnp.min(jnp.where(rm == m, rowid, _IBIG))
        rm = jnp.where(rowid == r, _NEG, rm)
        sel_lane = jnp.where(lane64 == t, r, sel_lane)
        sel_col = jnp.where(sub64 == t, r, sel_col)
        row = subj_ref[pl.ds(r, 1), :]
        gath = jnp.where(sub64 == t, row, gath)
        return rm, sel_lane, sel_col, gath

    carry = (rowmax,
             jnp.zeros((1, _TAKE), jnp.int32),
             jnp.zeros((_TAKE, 1), jnp.int32),
             jnp.zeros((_TAKE, _H), jnp.float32))
    _, sel_lane, sel_col, gath = jax.lax.fori_loop(0, _TAKE, select_row, carry)

    L = jax.lax.dot_general(gath, obj, (((1,), (1,)), ((), ())),
                            preferred_element_type=jnp.float32)
    L_ref[...] = L
    lrowmax = jnp.max(L, axis=1, keepdims=True)
    col_iota = jax.lax.broadcasted_iota(jnp.int32, (1, _N), 1)

    def pop(t, carry):
        lrm, scores, rvec, cvec = carry
        m = jnp.max(lrm)
        ro = jnp.min(jnp.where(lrm == m, sel_col, _IBIG))
        tloc = jnp.min(jnp.where((lrm == m) & (sel_col == ro), sub64, _IBIG))
        lrow = L_ref[pl.ds(tloc, 1), :]
        c = jnp.min(jnp.where(lrow == m, col_iota, _IBIG))
        lrow2 = jnp.where(col_iota == c, _NEG, lrow)
        L_ref[pl.ds(tloc, 1), :] = lrow2
        lrm = jnp.where(sub64 == tloc, jnp.max(lrow2), lrm)
        scores = jnp.where(lane64 == t, m, scores)
        rvec = jnp.where(lane64 == t, ro, rvec)
        cvec = jnp.where(lane64 == t, c, cvec)
        return lrm, scores, rvec, cvec

    carry = (lrowmax,
             jnp.full((1, _TAKE), _NEG, jnp.float32),
             jnp.zeros((1, _TAKE), jnp.int32),
             jnp.zeros((1, _TAKE), jnp.int32))
    _, scores, rvec, cvec = jax.lax.fori_loop(0, _TAKE, pop, carry)

    pairs_ref[0, 0:1, :] = rvec
    pairs_ref[0, 1:2, :] = cvec
    scores_ref[0, 0:1, :] = jax.nn.sigmoid(scores)


@jax.jit
def kernel(class_logits, proposals, W1s, b1s, W2s, b2s, W1o, b1o, W2o, b2o):
    del proposals
    B = class_logits.shape[0]
    b1s2 = b1s.reshape(1, _H)
    b2s2 = b2s.reshape(1, _H)
    b1o2 = b1o.reshape(1, _H)
    b2o2 = b2o.reshape(1, _H)

    full = lambda shape: pl.BlockSpec(shape, lambda b: (0,) * len(shape))
    pairs2, scores = pl.pallas_call(
        _relpn_kernel,
        grid=(B,),
        in_specs=[
            pl.BlockSpec((1, _N, _C), lambda b: (b, 0, 0)),
            full((_C, _H)), full((1, _H)), full((_H, _H)), full((1, _H)),
            full((_C, _H)), full((1, _H)), full((_H, _H)), full((1, _H)),
        ],
        out_specs=[
            pl.BlockSpec((1, 2, _TAKE), lambda b: (b, 0, 0)),
            pl.BlockSpec((1, 1, _TAKE), lambda b: (b, 0, 0)),
        ],
        out_shape=[
            jax.ShapeDtypeStruct((B, 2, _TAKE), jnp.int32),
            jax.ShapeDtypeStruct((B, 1, _TAKE), jnp.float32),
        ],
        scratch_shapes=[
            pltpu.VMEM((_N, _H), jnp.float32),
            pltpu.VMEM((_TAKE, _N), jnp.float32),
        ],
        compiler_params=pltpu.CompilerParams(
            dimension_semantics=("parallel",),
        ),
    )(class_logits, W1s, b1s2, W2s, b2s2, W1o, b1o2, W2o, b2o2)

    pairs = jnp.swapaxes(pairs2, 1, 2)
    return pairs, scores[:, 0, :]
